# baseline (speedup 1.0000x reference)
import functools
import jax
import jax.numpy as jnp
from jax import lax
from jax.experimental import pallas as pl
from jax.experimental.pallas import tpu as pltpu, tpu_sc as plsc

_VOCAB = 1000
_HID = 1024
_MASK_TOKEN_ID = 3
_EPS = 1e-06
_MASK_RATIO_TRAIN = 0.12

_B = 4
_T = 2048
_NTOK = _B * _T
_L = 16
_CHUNK = 16
_SLICES = _HID // _L
_NBUF = 6
_LEAD = 4


def _rsqrt_newton(x):
    i = plsc.bitcast(x, jnp.int32)
    i = jnp.int32(0x5F3759DF) - lax.shift_right_logical(i, 1)
    y = plsc.bitcast(i, jnp.float32)
    for _ in range(3):
        y = y * (jnp.float32(1.5) - jnp.float32(0.5) * x * y * y)
    return y


def _make_kernel():
    info = plsc.get_sparse_core_info()
    nc, ns = info.num_cores, info.num_subcores
    nw = nc * ns
    tok_per_w = _NTOK // nw
    w_per_row = _T // tok_per_w
    nchunk = tok_per_w // _CHUNK
    mesh = plsc.VectorSubcoreMesh(core_axis_name="c", subcore_axis_name="s")

    @functools.partial(
        pl.kernel,
        mesh=mesh,
        compiler_params=pltpu.CompilerParams(needs_layout_passes=False),
        out_type=jax.ShapeDtypeStruct((_NTOK, _HID), jnp.float32),
        scratch_types=[
            pltpu.VMEM((_T,), jnp.int32),
            pltpu.VMEM((_T,), jnp.int32),
            pltpu.VMEM((_HID,), jnp.float32),
            pltpu.VMEM((_CHUNK, _L), jnp.float32),
            pltpu.VMEM((2 * _L,), jnp.float32),
            [pltpu.VMEM((_CHUNK, _HID), jnp.float32) for _ in range(_NBUF)],
            [pltpu.VMEM((_CHUNK,), jnp.int32) for _ in range(_NBUF)],
            [pltpu.SemaphoreType.DMA for _ in range(_NBUF)],
            [pltpu.SemaphoreType.DMA for _ in range(_NBUF)],
        ],
    )
    def k(ids_hbm, am_hbm, table_hbm, lnw_hbm, out_hbm,
          ids_v, am_v, lnw_v, ssq_v, fac_v, rows, idxs, gsem, osem):
        cid = lax.axis_index("c")
        sid = lax.axis_index("s")
        wid = sid * nc + cid
        row = wid // w_per_row
        local = (wid % w_per_row) * tok_per_w
        base = row * _T

        pltpu.sync_copy(ids_hbm.at[pl.ds(base, _T)], ids_v)

        lanes = lax.iota(jnp.int32, _L)
        zero16 = jnp.zeros((_L,), jnp.float32)

        def issue_gather(c, p):
            st = local + c * _CHUNK
            idxs[p][...] = ids_v[pl.ds(st, _CHUNK)]
            return pltpu.async_copy(table_hbm.at[idxs[p]], rows[p], gsem[p])

        gh = [None] * nchunk
        for c in range(_LEAD):
            gh[c] = issue_gather(c, c)

        pltpu.sync_copy(am_hbm.at[pl.ds(base, _T)], am_v)
        pltpu.sync_copy(lnw_hbm, lnw_v)

        def stats_body(i, carry):
            nm, na = carry
            ids16 = ids_v[pl.ds(i * _L, _L)]
            am16 = am_v[pl.ds(i * _L, _L)]
            one = jnp.ones((_L,), jnp.float32)
            nm = nm + jnp.where(ids16 == _MASK_TOKEN_ID, one, zero16)
            na = na + jnp.where(am16 > 0, one, zero16)
            return nm, na

        nm_v, na_v = lax.fori_loop(0, _T // _L, stats_body, (zero16, zero16))
        fac_v[pl.ds(0, _L)] = nm_v
        fac_v[pl.ds(_L, _L)] = na_v
        n_mask = zero16
        n_att = zero16
        for j in range(_L):
            jv = jnp.full((_L,), j, jnp.int32)
            n_mask = n_mask + plsc.load_gather(fac_v, [jv])
            n_att = n_att + plsc.load_gather(fac_v, [jv + jnp.int32(_L)])
        scale = jnp.float32(1.0 - _MASK_RATIO_TRAIN) / (
            jnp.float32(1.0) - n_mask / n_att)
        s2_over_h = scale * scale * jnp.float32(1.0 / _HID)

        oh = [None] * _NBUF
        for c in range(nchunk):
            p = c % _NBUF
            gh[c].wait()
            rp = rows[p]

            def ssq_body(j, accs, rp=rp):
                sl = pl.ds(j * _L, _L)
                return tuple(
                    accs[t] + rp[t, sl] * rp[t, sl] for t in range(_CHUNK))

            accs = lax.fori_loop(0, _SLICES, ssq_body, (zero16,) * _CHUNK)
            for t in range(_CHUNK):
                ssq_v[t, :] = accs[t]
            m = zero16
            for j in range(_L):
                jv = jnp.full((_L,), j, jnp.int32)
                m = m + plsc.load_gather(ssq_v, [lanes, jv])

            ids16 = ids_v[pl.ds(local + c * _CHUNK, _L)]
            am16 = am_v[pl.ds(local + c * _CHUNK, _L)]
            var_eps = m * s2_over_h + jnp.float32(_EPS)
            f = scale * _rsqrt_newton(var_eps)
            f = jnp.where(ids16 == _MASK_TOKEN_ID, zero16, f)
            f = jnp.where(am16 > 0, f, zero16)
            fac_v[pl.ds(_L, _L)] = f

            fs = tuple(
                plsc.load_gather(fac_v, [jnp.full((_L,), _L + t, jnp.int32)])
                for t in range(_CHUNK))

            def scale_j(j, _, rp=rp, fs=fs):
                sl = pl.ds(j * _L, _L)
                lnwj = lnw_v[sl]
                for t in range(_CHUNK):
                    rp[t, sl] = rp[t, sl] * fs[t] * lnwj
                return 0

            lax.fori_loop(0, _SLICES, scale_j, 0)

            oh[p] = pltpu.async_copy(
                rp, out_hbm.at[pl.ds(base + local + c * _CHUNK, _CHUNK)],
                osem[p])
            nxt = c + _LEAD
            if nxt < nchunk:
                q = nxt % _NBUF
                if oh[q] is not None:
                    oh[q].wait()
                gh[nxt] = issue_gather(nxt, q)
        for p in range(_NBUF):
            if oh[p] is not None:
                oh[p].wait()

    return k


_kernel_call = _make_kernel()


def kernel(input_ids, attention_mask, emb_table, ln_weight):
    ids_flat = input_ids.reshape(_NTOK).astype(jnp.int32)
    am_flat = attention_mask.reshape(_NTOK).astype(jnp.int32)
    out = _kernel_call(ids_flat, am_flat, emb_table, ln_weight)
    return out.reshape(_B, _T, _HID)


# speedup vs baseline: 1.1507x; 1.1507x over previous
import functools
import jax
import jax.numpy as jnp
from jax import lax
from jax.experimental import pallas as pl
from jax.experimental.pallas import tpu as pltpu, tpu_sc as plsc

_VOCAB = 1000
_HID = 1024
_MASK_TOKEN_ID = 3
_EPS = 1e-06
_MASK_RATIO_TRAIN = 0.12

_B = 4
_T = 2048
_NTOK = _B * _T
_L = 16
_CHUNK = 32
_HALF = _CHUNK // 2
_SLICES = _HID // _L
_VPAD = 1024


def _rsqrt_newton(x):
    i = plsc.bitcast(x, jnp.int32)
    i = jnp.int32(0x5F3759DF) - lax.shift_right_logical(i, 1)
    y = plsc.bitcast(i, jnp.float32)
    for _ in range(3):
        y = y * (jnp.float32(1.5) - jnp.float32(0.5) * x * y * y)
    return y


def _make_kernel():
    info = plsc.get_sparse_core_info()
    nc, ns = info.num_cores, info.num_subcores
    nw = nc * ns
    tok_per_w = _NTOK // nw
    w_per_row = _T // tok_per_w
    nchunk = tok_per_w // _CHUNK
    v_per_s = _VPAD // ns
    mesh = plsc.VectorSubcoreMesh(core_axis_name="c", subcore_axis_name="s")

    @functools.partial(
        pl.kernel,
        mesh=mesh,
        compiler_params=pltpu.CompilerParams(needs_layout_passes=False),
        out_type=jax.ShapeDtypeStruct((_NTOK, _HID), jnp.float32),
        scratch_types=[
            pltpu.VMEM((_T,), jnp.int32),
            pltpu.VMEM((_T,), jnp.int32),
            pltpu.VMEM((_HID,), jnp.float32),
            pltpu.VMEM((_VPAD,), jnp.float32),
            pltpu.VMEM((_CHUNK, _L), jnp.float32),
            pltpu.VMEM((tok_per_w + _L,), jnp.float32),
            pltpu.VMEM((_CHUNK, _HID), jnp.float32),
            pltpu.VMEM((_CHUNK, _HID), jnp.float32),
            pltpu.VMEM((_CHUNK, _HID), jnp.float32),
            pltpu.VMEM((_CHUNK,), jnp.int32),
            pltpu.VMEM((_CHUNK,), jnp.int32),
            pltpu.VMEM((_CHUNK,), jnp.int32),
            pltpu.VMEM_SHARED((_VPAD,), jnp.float32),
            pltpu.SemaphoreType.DMA,
            pltpu.SemaphoreType.DMA,
            pltpu.SemaphoreType.DMA,
            pltpu.SemaphoreType.DMA,
            pltpu.SemaphoreType.DMA,
            pltpu.SemaphoreType.DMA,
        ],
    )
    def k(ids_hbm, am_hbm, table_hbm, lnw_hbm, out_hbm,
          ids_v, am_v, lnw_v, ssq_all_v, ssq_v, fac_v,
          rows0, rows1, rows2, idx0, idx1, idx2, ssq_sh,
          g0, g1, g2, o0, o1, o2):
        rows = (rows0, rows1, rows2)
        idxs = (idx0, idx1, idx2)
        gsem = (g0, g1, g2)
        osem = (o0, o1, o2)

        cid = lax.axis_index("c")
        sid = lax.axis_index("s")
        wid = sid * nc + cid
        row = wid // w_per_row
        local = (wid % w_per_row) * tok_per_w
        base = row * _T

        pltpu.sync_copy(ids_hbm.at[pl.ds(base, _T)], ids_v)

        lanes = lax.iota(jnp.int32, _L)
        zero16 = jnp.zeros((_L,), jnp.float32)

        def issue_gather(c, p):
            st = local + c * _CHUNK
            for g in range(_CHUNK // _L):
                idxs[p][pl.ds(g * _L, _L)] = ids_v[pl.ds(st + g * _L, _L)]
            return pltpu.async_copy(table_hbm.at[idxs[p]], rows[p], gsem[p])

        gh = [None] * nchunk
        gh[0] = issue_gather(0, 0)

        vbase = sid * v_per_s
        for q in range(v_per_s // _CHUNK):
            for g in range(_CHUNK // _L):
                vrow = vbase + jnp.int32(q * _CHUNK + g * _L) + lanes
                idxs[1 + q][pl.ds(g * _L, _L)] = jnp.minimum(
                    vrow, jnp.int32(_VOCAB - 1))
        p0h = [pltpu.async_copy(table_hbm.at[idxs[1 + q]], rows[1 + q],
                                gsem[1 + q])
               for q in range(v_per_s // _CHUNK)]

        pltpu.sync_copy(am_hbm.at[pl.ds(base, _T)], am_v)
        pltpu.sync_copy(lnw_hbm, lnw_v)

        def stats_body(i, carry):
            nm, na = carry
            ids16 = ids_v[pl.ds(i * _L, _L)]
            am16 = am_v[pl.ds(i * _L, _L)]
            one = jnp.ones((_L,), jnp.float32)
            nm = nm + jnp.where(ids16 == _MASK_TOKEN_ID, one, zero16)
            na = na + jnp.where(am16 > 0, one, zero16)
            return nm, na

        nm_v, na_v = lax.fori_loop(0, _T // _L, stats_body, (zero16, zero16))
        fac_v[pl.ds(0, _L)] = nm_v
        fac_v[pl.ds(_L, _L)] = na_v
        n_mask = zero16
        n_att = zero16
        for j in range(_L):
            jv = jnp.full((_L,), j, jnp.int32)
            n_mask = n_mask + plsc.load_gather(fac_v, [jv])
            n_att = n_att + plsc.load_gather(fac_v, [jv + jnp.int32(_L)])
        scale = jnp.float32(1.0 - _MASK_RATIO_TRAIN) / (
            jnp.float32(1.0) - n_mask / n_att)
        s2_over_h = scale * scale * jnp.float32(1.0 / _HID)

        for q in range(v_per_s // _CHUNK):
            p0h[q].wait()

            for h in range(_CHUNK // _HALF):
                def p0_body(j, accs, q=q, h=h):
                    sl = pl.ds(j * _L, _L)
                    return tuple(
                        accs[t] + rows[1 + q][h * _HALF + t, sl]
                        * rows[1 + q][h * _HALF + t, sl]
                        for t in range(_HALF))

                accs = lax.fori_loop(0, _SLICES, p0_body,
                                     (zero16,) * _HALF)
                for t in range(_HALF):
                    ssq_v[h * _HALF + t, :] = accs[t]

            if q == 0:
                gh[1] = issue_gather(1, 1)

            for g in range(_CHUNK // _L):
                m = zero16
                ridx = lanes + jnp.int32(g * _L)
                for j in range(_L):
                    jv = jnp.full((_L,), j, jnp.int32)
                    m = m + plsc.load_gather(ssq_v, [ridx, jv])
                ssq_all_v[pl.ds(vbase + q * _CHUNK + g * _L, _L)] = m

        pltpu.sync_copy(ssq_all_v.at[pl.ds(vbase, v_per_s)],
                        ssq_sh.at[pl.ds(vbase, v_per_s)])
        plsc.subcore_barrier()
        pltpu.sync_copy(ssq_sh, ssq_all_v)

        for g in range(tok_per_w // _L):
            ids16 = ids_v[pl.ds(local + g * _L, _L)]
            am16 = am_v[pl.ds(local + g * _L, _L)]
            sv = plsc.load_gather(ssq_all_v, [ids16])
            var_eps = sv * s2_over_h + jnp.float32(_EPS)
            f = scale * _rsqrt_newton(var_eps)
            f = jnp.where(ids16 == _MASK_TOKEN_ID, zero16, f)
            f = jnp.where(am16 > 0, f, zero16)
            fac_v[pl.ds(_L + g * _L, _L)] = f

        oh = [None, None, None]
        for c in range(nchunk):
            p = c % 3
            gh[c].wait()
            rp = rows[p]

            for h in range(_CHUNK // _HALF):
                fs = tuple(
                    plsc.load_gather(
                        fac_v,
                        [jnp.full((_L,), _L + c * _CHUNK + h * _HALF + t,
                                  jnp.int32)])
                    for t in range(_HALF))

                def scale_j(j, _, rp=rp, fs=fs, h=h):
                    sl = pl.ds(j * _L, _L)
                    lnwj = lnw_v[sl]
                    for t in range(_HALF):
                        rp[h * _HALF + t, sl] = (
                            rp[h * _HALF + t, sl] * fs[t] * lnwj)
                    return 0

                lax.fori_loop(0, _SLICES, scale_j, 0)

            oh[p] = pltpu.async_copy(
                rp, out_hbm.at[pl.ds(base + local + c * _CHUNK, _CHUNK)],
                osem[p])
            nxt = c + 2
            if nxt < nchunk:
                q = nxt % 3
                if oh[q] is not None:
                    oh[q].wait()
                gh[nxt] = issue_gather(nxt, q)
        for p in range(3):
            if oh[p] is not None:
                oh[p].wait()

    return k


_kernel_call = _make_kernel()


def kernel(input_ids, attention_mask, emb_table, ln_weight):
    ids_flat = input_ids.reshape(_NTOK).astype(jnp.int32)
    am_flat = attention_mask.reshape(_NTOK).astype(jnp.int32)
    out = _kernel_call(ids_flat, am_flat, emb_table, ln_weight)
    return out.reshape(_B, _T, _HID)
